# baseline (speedup 1.0000x reference)
import functools

import jax
import jax.numpy as jnp
from jax import lax
from jax.experimental import pallas as pl
from jax.experimental.pallas import tpu as pltpu
from jax.experimental.pallas import tpu_sc as plsc

N_SC = 2
N_TILES = 16
NW = N_SC * N_TILES
CHUNK = 128
LANES = 16
BN = 1000
EPS = 1e-5
GROUPS = 32


NBUF = 2
IBLK = 40


@functools.lru_cache(maxsize=None)
def _make_sc_aggregate(n_nodes: int, hf: int, nch: int):
    npk = n_nodes // 2
    rpt = (npk // N_TILES) // 8 * 8
    rem = npk - N_TILES * rpt
    nchp = -(-nch // 8) * 8
    cpt = 2 * nchp
    nblk = cpt // IBLK
    assert cpt % IBLK == 0 and IBLK % 8 == 0
    mesh = plsc.VectorSubcoreMesh(core_axis_name="c", subcore_axis_name="s")

    @functools.partial(
        pl.kernel,
        out_type=jax.ShapeDtypeStruct((2 * npk, 2 * hf), jnp.float32),
        mesh=mesh,
        scratch_types=[
            pltpu.VMEM((IBLK, CHUNK), jnp.int32),
            pltpu.VMEM((IBLK, CHUNK), jnp.int32),
            pltpu.VMEM((IBLK, CHUNK), jnp.float32),
            [pltpu.VMEM((CHUNK, 2 * hf), jnp.float32)] * NBUF,
            [pltpu.VMEM((CHUNK,), jnp.int32)] * NBUF,
            pltpu.VMEM((CHUNK,), jnp.int32),
            pltpu.VMEM_SHARED((npk, 2 * hf), jnp.float32),
            pltpu.VMEM_SHARED((npk, 2 * hf), jnp.float32),
            pltpu.SemaphoreType.DMA,
            pltpu.SemaphoreType.DMA,
        ],
    )
    def sc_agg(spk_hbm, src_hbm, tgt_hbm, w_hbm, zeros_hbm, out_hbm,
               src_v, tgt_v, w_v, gbufs, gidx, tidx, s_sh, agg_sh,
               gsem, isem):
        c = lax.axis_index("c")
        s = lax.axis_index("s")

        pltpu.sync_copy(spk_hbm.at[pl.ds(c * npk + s * rpt, rpt)],
                        s_sh.at[pl.ds(s * rpt, rpt)])
        pltpu.sync_copy(zeros_hbm.at[pl.ds(s * rpt, rpt)],
                        agg_sh.at[pl.ds(s * rpt, rpt)])
        if rem:
            @pl.when(s == 0)
            def _():
                pltpu.sync_copy(
                    spk_hbm.at[pl.ds(c * npk + N_TILES * rpt, rem)],
                    s_sh.at[pl.ds(N_TILES * rpt, rem)])
                pltpu.sync_copy(zeros_hbm.at[pl.ds(N_TILES * rpt, rem)],
                                agg_sh.at[pl.ds(N_TILES * rpt, rem)])
        plsc.subcore_barrier()

        def prep_gidx(r, b):
            def gb(g, carry):
                sl = pl.ds(g * LANES, LANES)
                gidx[b][sl] = lax.shift_right_logical(
                    src_v[r, sl], jnp.int32(1))
                return carry
            lax.fori_loop(0, CHUNK // LANES, gb, 0, unroll=False)

        def g_start(b):
            pltpu.async_copy(s_sh.at[gidx[b]], gbufs[b], gsem)

        def g_wait(b):
            pltpu.make_async_copy(s_sh.at[gidx[b]], gbufs[b], gsem).wait()

        zvec = jnp.zeros((LANES,), jnp.float32)

        def blk_body(blk, carry):
            base = s * cpt + blk * IBLK
            pltpu.async_copy(src_hbm.at[pl.ds(base, IBLK)], src_v, isem)
            pltpu.async_copy(tgt_hbm.at[pl.ds(base, IBLK)], tgt_v, isem)
            pltpu.async_copy(w_hbm.at[pl.ds(base, IBLK)], w_v, isem)
            pltpu.make_async_copy(src_hbm.at[pl.ds(base, IBLK)], src_v,
                                  isem).wait()
            pltpu.make_async_copy(tgt_hbm.at[pl.ds(base, IBLK)], tgt_v,
                                  isem).wait()
            pltpu.make_async_copy(w_hbm.at[pl.ds(base, IBLK)], w_v,
                                  isem).wait()

            for b in range(NBUF):
                prep_gidx(b, b)
                g_start(b)

            def pair_body(q, carry2):
                for half in range(NBUF):
                    r = q * NBUF + half
                    b = half
                    g_wait(b)

                    def grp_body(g, carry3):
                        row0 = g * LANES
                        sl16 = pl.ds(row0, LANES)
                        mv = w_v[r, sl16]
                        sv = src_v[r, sl16]
                        tv = tgt_v[r, sl16]
                        tg_half = (tv & 1) * (2 * hf // 2)
                        sr_half = (sv & 1) * (2 * hf // 2)
                        tidx[sl16] = lax.shift_right_logical(tv, 1)
                        for e in range(LANES):
                            row = row0 + e
                            m = lax.broadcast(mv[e], (LANES,))
                            soff = sr_half[e]
                            toff = tg_half[e]
                            vals = [gbufs[b][row, pl.ds(soff + j * LANES,
                                                        LANES)] * m
                                    for j in range(hf // LANES)]
                            for j in range(hf // LANES):
                                gbufs[b][row, pl.ds(toff + j * LANES,
                                                    LANES)] = vals[j]
                            ooff = (2 * hf // 2) - toff
                            for j in range(hf // LANES):
                                gbufs[b][row, pl.ds(ooff + j * LANES,
                                                    LANES)] = zvec
                        return carry3

                    lax.fori_loop(0, CHUNK // LANES, grp_body, 0,
                                  unroll=False)
                    pltpu.sync_copy(gbufs[b], agg_sh.at[tidx], add=True)

                    @pl.when(r + NBUF < IBLK)
                    def _():
                        prep_gidx(r + NBUF, b)
                        g_start(b)
                return carry2

            lax.fori_loop(0, IBLK // NBUF, pair_body, 0, unroll=False)
            return carry

        lax.fori_loop(0, nblk, blk_body, 0, unroll=False)
        plsc.subcore_barrier()

        pltpu.sync_copy(agg_sh.at[pl.ds(s * rpt, rpt)],
                        out_hbm.at[pl.ds(c * npk + s * rpt, rpt)])
        if rem:
            @pl.when(s == 0)
            def _():
                pltpu.sync_copy(
                    agg_sh.at[pl.ds(N_TILES * rpt, rem)],
                    out_hbm.at[pl.ds(c * npk + N_TILES * rpt, rem)])

    return sc_agg


def _sc_aggregate(slo, shi, srcp, tgtp, wp, zeros, n_nodes, hf, nch):
    k = _make_sc_aggregate(n_nodes, hf, nch)
    spk = jnp.concatenate([slo.reshape(n_nodes // 2, 2 * hf),
                           shi.reshape(n_nodes // 2, 2 * hf)], axis=0)
    parts = k(spk, srcp, tgtp, wp, zeros)
    return parts.reshape(2, n_nodes, hf)


def _bs_rows(shape):
    nd = len(shape)
    return pl.BlockSpec((BN,) + tuple(shape[1:]),
                        lambda i, _nd=nd: (i,) + (0,) * (_nd - 1))


def _bs_parts(shape):
    return pl.BlockSpec((2, BN) + tuple(shape[2:]),
                        lambda i: (0, i, 0))


def _bs_full(shape):
    nd = len(shape)
    return pl.BlockSpec(tuple(shape), lambda i, _nd=nd: (0,) * _nd)


def _row_call(body, outs, *args, specs):
    n = args[0].shape[1] if args[0].ndim == 3 else args[0].shape[0]
    return pl.pallas_call(
        body,
        grid=(n // BN,),
        in_specs=[s(a.shape) for a, s in zip(args, specs)],
        out_specs=[_bs_rows(o.shape) for o in outs],
        out_shape=[jax.ShapeDtypeStruct(o.shape, o.dtype) for o in outs],
    )(*args)


def _gn(xb, gm_ref, gamma_ref, beta_ref):
    m = jnp.dot(xb, gm_ref[...], preferred_element_type=jnp.float32)
    d = xb - m
    v = jnp.dot(d * d, gm_ref[...], preferred_element_type=jnp.float32)
    return d * lax.rsqrt(v + EPS) * gamma_ref[...] + beta_ref[...]


def _cat(a_ref):
    return jnp.concatenate([a_ref[0], a_ref[1]], axis=-1)


def _mm0(x, w0):
    n, f = x.shape[0], w0.shape[1]
    hf = f // 2

    def body(x_ref, w_ref, lo_ref, hi_ref):
        sres = jnp.dot(x_ref[...], w_ref[...],
                       preferred_element_type=jnp.float32)
        lo_ref[...] = sres[:, :hf]
        hi_ref[...] = sres[:, hf:]

    o = jax.ShapeDtypeStruct((n, hf), jnp.float32)
    return _row_call(body, [o, o], x, w0, specs=[_bs_rows, _bs_full])


def _step_first(parts, b0r, gm, gamr, betr, w1b):
    n, hf = parts.shape[1], parts.shape[2]
    f = 2 * hf

    def body(a_ref, b_ref, gm_ref, g_ref, be_ref, w_ref, h_ref, lo_ref,
             hi_ref):
        h = jax.nn.relu(_cat(a_ref) + b_ref[...])
        h_ref[...] = h
        hn = _gn(h, gm_ref, g_ref, be_ref)
        sres = jnp.dot(hn, w_ref[...], preferred_element_type=jnp.float32)
        lo_ref[...] = sres[:, :hf]
        hi_ref[...] = sres[:, hf:]

    o = jax.ShapeDtypeStruct((n, f), jnp.float32)
    oh = jax.ShapeDtypeStruct((n, hf), jnp.float32)
    return _row_call(body, [o, oh, oh], parts, b0r, gm, gamr, betr, w1b,
                     specs=[_bs_parts, _bs_full, _bs_full, _bs_full,
                            _bs_full, _bs_full])


def _step_mid(parts, h, b1r, gm, gamr, betr, w1b, w1a, coef, tval):
    n, hf = parts.shape[1], parts.shape[2]
    f = 2 * hf

    def body(a_ref, h_ref, b_ref, gm_ref, g_ref, be_ref, w_ref, wa_ref,
             k_ref, lo_ref, hi_ref):
        k = jax.nn.relu(_cat(a_ref) + b_ref[...])
        k_ref[...] = k
        u = h_ref[...] + coef * k
        hn = _gn(u, gm_ref, g_ref, be_ref)
        sres = (jnp.dot(hn, w_ref[...], preferred_element_type=jnp.float32)
                + tval * wa_ref[...])
        lo_ref[...] = sres[:, :hf]
        hi_ref[...] = sres[:, hf:]

    o = jax.ShapeDtypeStruct((n, f), jnp.float32)
    oh = jax.ShapeDtypeStruct((n, hf), jnp.float32)
    return _row_call(body, [o, oh, oh], parts, h, b1r, gm, gamr, betr, w1b,
                     w1a,
                     specs=[_bs_parts, _bs_rows, _bs_full, _bs_full, _bs_full,
                            _bs_full, _bs_full, _bs_full])


def _step_last(parts, h, k1, k2, k3, b1r, gm, gamr, betr, w1b, w1a, dt, tval):
    n, hf = parts.shape[1], parts.shape[2]
    f = 2 * hf

    def body(a_ref, h_ref, k1_ref, k2_ref, k3_ref, b_ref, gm_ref, g_ref,
             be_ref, w_ref, wa_ref, hn_ref, lo_ref, hi_ref):
        k4 = jax.nn.relu(_cat(a_ref) + b_ref[...])
        hnew = h_ref[...] + (dt / 6.0) * (k1_ref[...] + 2.0 * k2_ref[...]
                                          + 2.0 * k3_ref[...] + k4)
        hn_ref[...] = hnew
        g = _gn(hnew, gm_ref, g_ref, be_ref)
        sres = (jnp.dot(g, w_ref[...], preferred_element_type=jnp.float32)
                + tval * wa_ref[...])
        lo_ref[...] = sres[:, :hf]
        hi_ref[...] = sres[:, hf:]

    o = jax.ShapeDtypeStruct((n, f), jnp.float32)
    oh = jax.ShapeDtypeStruct((n, hf), jnp.float32)
    return _row_call(body, [o, oh, oh], parts, h, k1, k2, k3, b1r, gm, gamr,
                     betr, w1b, w1a,
                     specs=[_bs_parts, _bs_rows, _bs_rows, _bs_rows, _bs_rows,
                            _bs_full, _bs_full, _bs_full, _bs_full, _bs_full,
                            _bs_full])


def _step_final(parts, h, k1, k2, k3, b1r, w2, dt):
    n = parts.shape[1]
    hf = w2.shape[1] // 2

    def body(a_ref, h_ref, k1_ref, k2_ref, k3_ref, b_ref, w2_ref, lo_ref,
             hi_ref):
        k4 = jax.nn.relu(_cat(a_ref) + b_ref[...])
        hnew = h_ref[...] + (dt / 6.0) * (k1_ref[...] + 2.0 * k2_ref[...]
                                          + 2.0 * k3_ref[...] + k4)
        sres = jnp.dot(hnew, w2_ref[...], preferred_element_type=jnp.float32)
        lo_ref[...] = sres[:, :hf]
        hi_ref[...] = sres[:, hf:]

    oh = jax.ShapeDtypeStruct((n, hf), jnp.float32)
    return _row_call(body, [oh, oh], parts, h, k1, k2, k3, b1r, w2,
                     specs=[_bs_parts, _bs_rows, _bs_rows, _bs_rows, _bs_rows,
                            _bs_full, _bs_full])


def _softmax_out(parts, b2r):
    n = parts.shape[1]
    nclass = b2r.shape[1]

    def body(a_ref, b_ref, o_ref):
        z = (a_ref[0] + a_ref[1])[:, :nclass] + b_ref[...]
        m = jnp.max(z, axis=1, keepdims=True)
        e = jnp.exp(z - m)
        lse = jnp.log(jnp.sum(e, axis=1, keepdims=True))
        o_ref[...] = z - m - lse

    o = jax.ShapeDtypeStruct((n, nclass), jnp.float32)
    return _row_call(body, [o], parts, b2r, specs=[_bs_parts, _bs_full])[0]


def kernel(x, src, tgt, Mtgt, W0, b0, gamma, beta, W1, b1, W2, b2):
    n, f = x.shape
    e = src.shape[0]
    nclass = W2.shape[1]
    dt = 1.0 / 4.0

    nch = -(-e // (CHUNK * NW))
    pad = nch * CHUNK * NW - e
    nchp = -(-nch // IBLK) * IBLK

    def _slab(a, fill):
        a = jnp.concatenate([a, jnp.full((pad,), fill, a.dtype)])
        a = a.reshape(NW, nch, CHUNK)
        a = jnp.pad(a, ((0, 0), (0, nchp - nch), (0, 0)))
        return a.reshape(NW * nchp, CHUNK)

    srcp = _slab(src, 0)
    tgtp = _slab(tgt, 0)
    wp = _slab(Mtgt, 0.0)
    zeros_h = jnp.zeros((n // 2, f), jnp.float32)
    w2p = jnp.pad(W2, ((0, 0), (0, f - nclass)))

    gidx = jnp.arange(f) // (f // GROUPS)
    gm = jnp.where(gidx[:, None] == gidx[None, :],
                   jnp.float32(GROUPS / f), 0.0)
    b0r = b0.reshape(1, f)
    b1r = b1.reshape(1, f)
    b2r = b2.reshape(1, nclass)
    gamr = gamma.reshape(1, f)
    betr = beta.reshape(1, f)
    w1a = W1[0:1, :]
    w1b = W1[1:, :]

    def agg(s_pair):
        return _sc_aggregate(s_pair[0], s_pair[1], srcp, tgtp, wp, zeros_h,
                             n, f // 2, nch)

    s = _mm0(x, W0)
    parts = agg(s)
    h, slo, shi = _step_first(parts, b0r, gm, gamr, betr, w1b)
    s = (slo, shi)

    for step in range(4):
        t0 = step * dt
        parts = agg(s)
        k1, slo, shi = _step_mid(parts, h, b1r, gm, gamr, betr, w1b, w1a,
                                 dt / 2.0, t0 + dt / 2.0)
        parts = agg((slo, shi))
        k2, slo, shi = _step_mid(parts, h, b1r, gm, gamr, betr, w1b, w1a,
                                 dt / 2.0, t0 + dt / 2.0)
        parts = agg((slo, shi))
        k3, slo, shi = _step_mid(parts, h, b1r, gm, gamr, betr, w1b, w1a,
                                 dt, t0 + dt)
        parts = agg((slo, shi))
        if step < 3:
            h, slo, shi = _step_last(parts, h, k1, k2, k3, b1r, gm, gamr,
                                     betr, w1b, w1a, dt, t0 + dt)
            s = (slo, shi)
        else:
            sF = _step_final(parts, h, k1, k2, k3, b1r, w2p, dt)

    parts = agg(sF)
    return _softmax_out(parts, b2r)


# speedup vs baseline: 1.0557x; 1.0557x over previous
import functools

import jax
import jax.numpy as jnp
from jax import lax
from jax.experimental import pallas as pl
from jax.experimental.pallas import tpu as pltpu
from jax.experimental.pallas import tpu_sc as plsc

N_SC = 2
N_TILES = 16
NW = N_SC * N_TILES
CHUNK = 128
LANES = 16
BN = 1000
EPS = 1e-5
GROUPS = 32


NBUF = 2
IBLK = 40


@functools.lru_cache(maxsize=None)
def _make_sc_aggregate(n_nodes: int, feat: int, nch: int):
    rpt = (n_nodes // N_TILES) // 8 * 8
    rem = n_nodes - N_TILES * rpt
    nchp = -(-nch // IBLK) * IBLK
    nblk = nchp // IBLK
    mesh = plsc.VectorSubcoreMesh(core_axis_name="c", subcore_axis_name="s")

    @functools.partial(
        pl.kernel,
        out_type=jax.ShapeDtypeStruct((2 * n_nodes, feat), jnp.float32),
        mesh=mesh,
        scratch_types=[
            pltpu.VMEM((IBLK, CHUNK), jnp.int32),
            pltpu.VMEM((IBLK, CHUNK), jnp.int32),
            pltpu.VMEM((IBLK, CHUNK), jnp.float32),
            [pltpu.VMEM((CHUNK, feat), jnp.float32)] * NBUF,
            pltpu.VMEM_SHARED((n_nodes, feat), jnp.float32),
            pltpu.SemaphoreType.DMA,
            pltpu.SemaphoreType.DMA,
        ],
    )
    def sc_agg(h_hbm, src_hbm, tgt_hbm, w_hbm, zeros_hbm, out_hbm,
               src_v, tgt_v, w_v, gbufs, agg_sh, gsem, isem):
        c = lax.axis_index("c")
        s = lax.axis_index("s")
        wid = s * N_SC + c

        pltpu.sync_copy(zeros_hbm.at[pl.ds(s * rpt, rpt)],
                        agg_sh.at[pl.ds(s * rpt, rpt)])
        if rem:
            @pl.when(s == 0)
            def _():
                pltpu.sync_copy(zeros_hbm.at[pl.ds(N_TILES * rpt, rem)],
                                agg_sh.at[pl.ds(N_TILES * rpt, rem)])
        plsc.subcore_barrier()

        def g_start(r, b):
            pltpu.async_copy(h_hbm.at[src_v.at[r]], gbufs[b], gsem)

        def g_wait(r, b):
            pltpu.make_async_copy(h_hbm.at[src_v.at[r]], gbufs[b], gsem).wait()

        def blk_body(blk, carry):
            base = wid * nchp + blk * IBLK
            pltpu.async_copy(src_hbm.at[pl.ds(base, IBLK)], src_v, isem)
            pltpu.async_copy(tgt_hbm.at[pl.ds(base, IBLK)], tgt_v, isem)
            pltpu.async_copy(w_hbm.at[pl.ds(base, IBLK)], w_v, isem)
            pltpu.make_async_copy(src_hbm.at[pl.ds(base, IBLK)], src_v,
                                  isem).wait()
            pltpu.make_async_copy(tgt_hbm.at[pl.ds(base, IBLK)], tgt_v,
                                  isem).wait()
            pltpu.make_async_copy(w_hbm.at[pl.ds(base, IBLK)], w_v,
                                  isem).wait()

            for b in range(NBUF):
                g_start(b, b)

            def pair_body(q, carry2):
                for half in range(NBUF):
                    r = q * NBUF + half
                    b = half
                    g_wait(r, b)

                    def grp_body(g, carry3):
                        row0 = g * LANES
                        mv = w_v[r, pl.ds(row0, LANES)]
                        for e in range(LANES):
                            row = row0 + e
                            m = lax.broadcast(mv[e], (LANES,))
                            for j in range(feat // LANES):
                                sl = pl.ds(j * LANES, LANES)
                                gbufs[b][row, sl] = gbufs[b][row, sl] * m
                        return carry3

                    lax.fori_loop(0, CHUNK // LANES, grp_body, 0,
                                  unroll=False)
                    pltpu.sync_copy(gbufs[b], agg_sh.at[tgt_v.at[r]],
                                    add=True)

                    @pl.when(r + NBUF < IBLK)
                    def _():
                        g_start(r + NBUF, b)
                return carry2

            lax.fori_loop(0, IBLK // NBUF, pair_body, 0, unroll=False)
            return carry

        lax.fori_loop(0, nblk, blk_body, 0, unroll=False)
        plsc.subcore_barrier()

        pltpu.sync_copy(agg_sh.at[pl.ds(s * rpt, rpt)],
                        out_hbm.at[pl.ds(c * n_nodes + s * rpt, rpt)])
        if rem:
            @pl.when(s == 0)
            def _():
                pltpu.sync_copy(
                    agg_sh.at[pl.ds(N_TILES * rpt, rem)],
                    out_hbm.at[pl.ds(c * n_nodes + N_TILES * rpt, rem)])

    return sc_agg


def _sc_aggregate(h, srcp, tgtp, wp, zeros, n_nodes, feat, nch):
    k = _make_sc_aggregate(n_nodes, feat, nch)
    parts = k(h, srcp, tgtp, wp, zeros)
    return parts.reshape(2, n_nodes, feat)


def _bs_rows(shape):
    nd = len(shape)
    return pl.BlockSpec((BN,) + tuple(shape[1:]),
                        lambda i, _nd=nd: (i,) + (0,) * (_nd - 1))


def _bs_parts(shape):
    return pl.BlockSpec((2, BN) + tuple(shape[2:]),
                        lambda i: (0, i, 0))


def _bs_full(shape):
    nd = len(shape)
    return pl.BlockSpec(tuple(shape), lambda i, _nd=nd: (0,) * _nd)


def _row_call(body, outs, *args, specs):
    n = args[0].shape[1] if args[0].ndim == 3 else args[0].shape[0]
    return pl.pallas_call(
        body,
        grid=(n // BN,),
        in_specs=[s(a.shape) for a, s in zip(args, specs)],
        out_specs=[_bs_rows(o.shape) for o in outs],
        out_shape=[jax.ShapeDtypeStruct(o.shape, o.dtype) for o in outs],
    )(*args)


def _gn(xb, gm_ref, gamma_ref, beta_ref):
    m = jnp.dot(xb, gm_ref[...], preferred_element_type=jnp.float32)
    d = xb - m
    v = jnp.dot(d * d, gm_ref[...], preferred_element_type=jnp.float32)
    return d * lax.rsqrt(v + EPS) * gamma_ref[...] + beta_ref[...]


def _mm0(x, w0):
    def body(x_ref, w_ref, o_ref):
        o_ref[...] = jnp.dot(x_ref[...], w_ref[...],
                             preferred_element_type=jnp.float32)
    o = jax.ShapeDtypeStruct((x.shape[0], w0.shape[1]), jnp.float32)
    return _row_call(body, [o], x, w0, specs=[_bs_rows, _bs_full])[0]


def _step_first(parts, b0r, gm, gamr, betr, w1b):
    n, f = parts.shape[1], parts.shape[2]

    def body(a_ref, b_ref, gm_ref, g_ref, be_ref, w_ref, h_ref, s_ref):
        h = jax.nn.relu(a_ref[0] + a_ref[1] + b_ref[...])
        h_ref[...] = h
        hn = _gn(h, gm_ref, g_ref, be_ref)
        s_ref[...] = jnp.dot(hn, w_ref[...], preferred_element_type=jnp.float32)

    o = jax.ShapeDtypeStruct((n, f), jnp.float32)
    return _row_call(body, [o, o], parts, b0r, gm, gamr, betr, w1b,
                     specs=[_bs_parts, _bs_full, _bs_full, _bs_full,
                            _bs_full, _bs_full])


def _step_mid(parts, h, b1r, gm, gamr, betr, w1b, w1a, coef, tval):
    n, f = parts.shape[1], parts.shape[2]

    def body(a_ref, h_ref, b_ref, gm_ref, g_ref, be_ref, w_ref, wa_ref,
             k_ref, s_ref):
        k = jax.nn.relu(a_ref[0] + a_ref[1] + b_ref[...])
        k_ref[...] = k
        u = h_ref[...] + coef * k
        hn = _gn(u, gm_ref, g_ref, be_ref)
        s_ref[...] = (jnp.dot(hn, w_ref[...], preferred_element_type=jnp.float32)
                      + tval * wa_ref[...])

    o = jax.ShapeDtypeStruct((n, f), jnp.float32)
    return _row_call(body, [o, o], parts, h, b1r, gm, gamr, betr, w1b, w1a,
                     specs=[_bs_parts, _bs_rows, _bs_full, _bs_full, _bs_full,
                            _bs_full, _bs_full, _bs_full])


def _step_last(parts, h, k1, k2, k3, b1r, gm, gamr, betr, w1b, w1a, dt, tval):
    n, f = parts.shape[1], parts.shape[2]

    def body(a_ref, h_ref, k1_ref, k2_ref, k3_ref, b_ref, gm_ref, g_ref,
             be_ref, w_ref, wa_ref, hn_ref, s_ref):
        k4 = jax.nn.relu(a_ref[0] + a_ref[1] + b_ref[...])
        hnew = h_ref[...] + (dt / 6.0) * (k1_ref[...] + 2.0 * k2_ref[...]
                                          + 2.0 * k3_ref[...] + k4)
        hn_ref[...] = hnew
        g = _gn(hnew, gm_ref, g_ref, be_ref)
        s_ref[...] = (jnp.dot(g, w_ref[...], preferred_element_type=jnp.float32)
                      + tval * wa_ref[...])

    o = jax.ShapeDtypeStruct((n, f), jnp.float32)
    return _row_call(body, [o, o], parts, h, k1, k2, k3, b1r, gm, gamr, betr,
                     w1b, w1a,
                     specs=[_bs_parts, _bs_rows, _bs_rows, _bs_rows, _bs_rows,
                            _bs_full, _bs_full, _bs_full, _bs_full, _bs_full,
                            _bs_full])


def _step_final(parts, h, k1, k2, k3, b1r, w2, dt):
    n = parts.shape[1]

    def body(a_ref, h_ref, k1_ref, k2_ref, k3_ref, b_ref, w2_ref, s_ref):
        k4 = jax.nn.relu(a_ref[0] + a_ref[1] + b_ref[...])
        hnew = h_ref[...] + (dt / 6.0) * (k1_ref[...] + 2.0 * k2_ref[...]
                                          + 2.0 * k3_ref[...] + k4)
        s_ref[...] = jnp.dot(hnew, w2_ref[...],
                             preferred_element_type=jnp.float32)

    o = jax.ShapeDtypeStruct((n, w2.shape[1]), jnp.float32)
    return _row_call(body, [o], parts, h, k1, k2, k3, b1r, w2,
                     specs=[_bs_parts, _bs_rows, _bs_rows, _bs_rows, _bs_rows,
                            _bs_full, _bs_full])[0]


def _softmax_out(parts, b2r):
    n = parts.shape[1]
    nclass = b2r.shape[1]

    def body(a_ref, b_ref, o_ref):
        z = (a_ref[0] + a_ref[1])[:, :nclass] + b_ref[...]
        m = jnp.max(z, axis=1, keepdims=True)
        e = jnp.exp(z - m)
        lse = jnp.log(jnp.sum(e, axis=1, keepdims=True))
        o_ref[...] = z - m - lse

    o = jax.ShapeDtypeStruct((n, nclass), jnp.float32)
    return _row_call(body, [o], parts, b2r, specs=[_bs_parts, _bs_full])[0]


def kernel(x, src, tgt, Mtgt, W0, b0, gamma, beta, W1, b1, W2, b2):
    n, f = x.shape
    e = src.shape[0]
    nclass = W2.shape[1]
    dt = 1.0 / 4.0

    nch = -(-e // (CHUNK * NW))
    pad = nch * CHUNK * NW - e
    nchp = -(-nch // IBLK) * IBLK

    def _slab(a, fill):
        a = jnp.concatenate([a, jnp.full((pad,), fill, a.dtype)])
        a = a.reshape(NW, nch, CHUNK)
        a = jnp.pad(a, ((0, 0), (0, nchp - nch), (0, 0)))
        return a.reshape(NW * nchp, CHUNK)

    srcp = _slab(src, 0)
    tgtp = _slab(tgt, 0)
    wp = _slab(Mtgt, 0.0)
    zeros_f = jnp.zeros((n, f), jnp.float32)
    w2p = jnp.pad(W2, ((0, 0), (0, f - nclass)))

    gidx = jnp.arange(f) // (f // GROUPS)
    gm = jnp.where(gidx[:, None] == gidx[None, :],
                   jnp.float32(GROUPS / f), 0.0)
    b0r = b0.reshape(1, f)
    b1r = b1.reshape(1, f)
    b2r = b2.reshape(1, nclass)
    gamr = gamma.reshape(1, f)
    betr = beta.reshape(1, f)
    w1a = W1[0:1, :]
    w1b = W1[1:, :]

    def agg(hh, feat, zz):
        return _sc_aggregate(hh, srcp, tgtp, wp, zz, n, feat, nch)

    s0 = _mm0(x, W0)
    parts = agg(s0, f, zeros_f)
    h, s = _step_first(parts, b0r, gm, gamr, betr, w1b)

    for step in range(4):
        t0 = step * dt
        parts = agg(s, f, zeros_f)
        k1, s = _step_mid(parts, h, b1r, gm, gamr, betr, w1b, w1a,
                          dt / 2.0, t0 + dt / 2.0)
        parts = agg(s, f, zeros_f)
        k2, s = _step_mid(parts, h, b1r, gm, gamr, betr, w1b, w1a,
                          dt / 2.0, t0 + dt / 2.0)
        parts = agg(s, f, zeros_f)
        k3, s = _step_mid(parts, h, b1r, gm, gamr, betr, w1b, w1a,
                          dt, t0 + dt)
        parts = agg(s, f, zeros_f)
        if step < 3:
            h, s = _step_last(parts, h, k1, k2, k3, b1r, gm, gamr, betr,
                              w1b, w1a, dt, t0 + dt)
        else:
            sF = _step_final(parts, h, k1, k2, k3, b1r, w2p, dt)

    parts = agg(sF, f, zeros_f)
    return _softmax_out(parts, b2r)
